# baseline (speedup 1.0000x reference)
import functools

import jax
import jax.numpy as jnp
from jax import lax
from jax.experimental import pallas as pl
from jax.experimental.pallas import tpu as pltpu
from jax.experimental.pallas import tpu_sc as plsc

BATCH = 16384
EMBED_DIM = 64
LANES = 16

_NC = 2
_NS = 16
_NTC = 7813
_TPW = 489
_NRING = 2
_NGRP = (_TPW + _NRING - 1) // _NRING
_WRING = 8
_ACH = 2048
_CPW = BATCH // _NS
_TSUB = 256


def _extract_scalar(ref, pos):
    return ref[pl.ds(pos, LANES)][0]


@functools.partial(
    pl.kernel,
    mesh=plsc.VectorSubcoreMesh(core_axis_name="c", subcore_axis_name="s"),
    out_type=(
        jax.ShapeDtypeStruct((EMBED_DIM, BATCH), jnp.float32),
        jax.ShapeDtypeStruct((EMBED_DIM, BATCH), jnp.float32),
        jax.ShapeDtypeStruct((_NC * BATCH * EMBED_DIM,), jnp.float32),
    ),
    scratch_types=[
        pltpu.VMEM((_ACH,), jnp.int32),
        pltpu.VMEM((BATCH + LANES,), jnp.int32),
        pltpu.VMEM((BATCH + LANES,), jnp.int32),
        pltpu.VMEM((BATCH + LANES,), jnp.int32),
        pltpu.VMEM((BATCH + LANES,), jnp.int32),
        pltpu.VMEM((512 + LANES,), jnp.int32),
        pltpu.VMEM((512 + LANES,), jnp.int32),
        pltpu.VMEM((_NRING, EMBED_DIM, 128), jnp.float32),
        pltpu.VMEM((_WRING * EMBED_DIM,), jnp.float32),
        pltpu.VMEM((_TSUB * EMBED_DIM,), jnp.float32),
        pltpu.VMEM((EMBED_DIM, _TSUB), jnp.float32),
        [pltpu.SemaphoreType.DMA] * _NRING,
        pltpu.SemaphoreType.DMA,
    ],
    compiler_params=pltpu.CompilerParams(needs_layout_passes=False),
)
def _gather_kernel(ids_cat, utabT, itabT, uoutT, ioutT, exch,
                   allids, clist_id, clist_pos, sort_id, sort_pos,
                   begin_v, end_v, stage, tmpc, tbuf, colsT, sems, sem_w):
    core = lax.axis_index("c")
    sid = lax.axis_index("s")
    ibase = core * BATCH
    obase = core * BATCH * EMBED_DIM
    iota = lax.iota(jnp.int32, LANES)
    ones = jnp.ones((LANES,), jnp.int32)

    zeros = jnp.zeros((LANES,), jnp.int32)

    def zero_body(b, acc):
        begin_v[pl.ds(b * LANES, LANES)] = zeros
        return acc

    lax.fori_loop(0, 512 // LANES, zero_body, jnp.int32(0), unroll=False)

    def macro_body(a, pos):
        pltpu.sync_copy(ids_cat.at[pl.ds(ibase + a * _ACH, _ACH)], allids)

        def scan_body(k, pos):
            v = allids[pl.ds(k * LANES, LANES)]
            m = ((v >> 7) & (_NS - 1)) == sid
            plsc.store_compressed(clist_id.at[pl.ds(pos, LANES)], v, mask=m)
            plsc.store_compressed(clist_pos.at[pl.ds(pos, LANES)],
                                  iota + (a * _ACH + k * LANES), mask=m)
            plsc.addupdate_scatter(begin_v, [v >> 11], ones, mask=m)
            return pos + plsc.all_reduce_population_count(m)[0]

        return lax.fori_loop(0, _ACH // LANES, scan_body, pos, unroll=False)

    nkeep = lax.fori_loop(0, BATCH // _ACH, macro_body, jnp.int32(0),
                          unroll=False)

    def prefix_body(b, carry):
        v = begin_v[pl.ds(b * LANES, LANES)]
        s = plsc.cumsum(v) + carry
        end_v[pl.ds(b * LANES, LANES)] = s
        begin_v[pl.ds(b * LANES, LANES)] = s - v
        return s[LANES - 1]

    lax.fori_loop(0, 512 // LANES, prefix_body, jnp.int32(0), unroll=False)

    lane0 = iota == 0
    nchunk = (nkeep + LANES - 1) // LANES

    def place_body(k, acc):
        v = clist_id[pl.ds(k * LANES, LANES)]
        p = clist_pos[pl.ds(k * LANES, LANES)]
        for lane in range(LANES):
            @pl.when(k * LANES + lane < nkeep)
            def _place():
                idv = v[lane]
                pos = p[lane]
                slot = idv >> 11
                dst = _extract_scalar(begin_v, slot)
                plsc.store_scatter(sort_id,
                                   [jnp.full((LANES,), dst, jnp.int32)],
                                   jnp.full((LANES,), idv, jnp.int32),
                                   mask=lane0)
                plsc.store_scatter(sort_pos,
                                   [jnp.full((LANES,), dst, jnp.int32)],
                                   jnp.full((LANES,), pos, jnp.int32),
                                   mask=lane0)
                plsc.addupdate_scatter(begin_v,
                                       [jnp.full((LANES,), slot, jnp.int32)],
                                       ones, mask=lane0)
        return acc

    lax.fori_loop(0, nchunk, place_body, jnp.int32(0), unroll=False)

    def enqueue(t, q):
        tcg = sid + t * _NS
        off = pl.multiple_of(jnp.minimum(tcg, _NTC - 1) * 128, 128)
        inb = tcg < _NTC

        @pl.when(jnp.logical_and(inb, core == 0))
        def _u():
            pltpu.async_copy(utabT.at[:, pl.ds(off, 128)],
                             stage.at[q], sems[q])

        @pl.when(jnp.logical_and(inb, core == 1))
        def _i():
            pltpu.async_copy(itabT.at[:, pl.ds(off, 128)],
                             stage.at[q], sems[q])

    dummy_tc = utabT.at[:, pl.ds(0, 128)]
    dummy_row = exch.at[pl.ds(0, EMBED_DIM)]

    for q in range(_NRING):
        enqueue(q, q)

    def extract_tc(t, q, wcnt):
        bp = _extract_scalar(end_v, jnp.maximum(t - 1, 0))
        b0 = lax.select(t > 0, bp, jnp.int32(0))
        b1 = _extract_scalar(end_v, t)

        def elem_body(e, wcnt):
            idv = _extract_scalar(sort_id, e)
            pos = _extract_scalar(sort_pos, e)
            cvec = jnp.full((LANES,), idv & 127, jnp.int32)
            ws = wcnt & (_WRING - 1)

            @pl.when(wcnt >= _WRING)
            def _wring():
                pltpu.make_async_copy(
                    dummy_row, tmpc.at[pl.ds(ws * EMBED_DIM, EMBED_DIM)],
                    sem_w).wait()

            def gat_body(b, acc):
                dvec = iota + b * LANES
                vv = plsc.load_gather(stage.at[q], [dvec, cvec])
                tmpc[pl.ds(ws * EMBED_DIM + b * LANES, LANES)] = vv
                return acc

            lax.fori_loop(0, EMBED_DIM // LANES, gat_body, jnp.int32(0),
                          unroll=False)
            pltpu.async_copy(tmpc.at[pl.ds(ws * EMBED_DIM, EMBED_DIM)],
                             exch.at[pl.ds(obase + pos * EMBED_DIM,
                                           EMBED_DIM)],
                             sem_w)
            return wcnt + 1

        return lax.fori_loop(b0, b1, elem_body, wcnt, unroll=False)

    def group_body(g, wcnt):
        for q in range(_NRING):
            t = g * _NRING + q

            @pl.when(t * _NS + sid < _NTC)
            def _wait():
                pltpu.make_async_copy(dummy_tc, stage.at[q], sems[q]).wait()

            wcnt = lax.cond(t < _TPW,
                            lambda w: extract_tc(t, q, w),
                            lambda w: w, wcnt)
            enqueue(t + _NRING, q)
        return wcnt

    wcnt = lax.fori_loop(0, _NGRP, group_body, jnp.int32(0), unroll=False)

    def drain_body(d, acc):
        @pl.when(d < wcnt)
        def _drain():
            pltpu.make_async_copy(dummy_row, tmpc.at[pl.ds(0, EMBED_DIM)],
                                  sem_w).wait()
        return acc

    lax.fori_loop(0, _WRING, drain_body, jnp.int32(0), unroll=False)

    plsc.subcore_barrier()
    cbase = sid * _CPW

    def sub_body(sb, acc):
        coff = pl.multiple_of(cbase + sb * _TSUB, 128)
        pltpu.sync_copy(
            exch.at[pl.ds(obase + coff * EMBED_DIM, _TSUB * EMBED_DIM)],
            tbuf)

        def tr_d(d, acc):
            dsp = jnp.full((LANES,), d, jnp.int32)

            def tr_b(b, acc):
                cv = iota + b * LANES
                v = plsc.load_gather(tbuf, [cv * EMBED_DIM + d])
                plsc.store_scatter(colsT, [dsp, cv], v)
                return acc

            return lax.fori_loop(0, _TSUB // LANES, tr_b, acc,
                                 unroll=False)

        lax.fori_loop(0, EMBED_DIM, tr_d, jnp.int32(0), unroll=False)
        csl = pl.ds(coff, _TSUB)

        @pl.when(core == 0)
        def _wu():
            pltpu.sync_copy(colsT, uoutT.at[:, csl])

        @pl.when(core == 1)
        def _wi():
            pltpu.sync_copy(colsT, ioutT.at[:, csl])
        return acc

    lax.fori_loop(0, _CPW // _TSUB, sub_body, jnp.int32(0), unroll=False)


def kernel(user_ids, item_ids, user_table, item_table):
    ids_cat = jnp.concatenate([user_ids, item_ids])
    uT, iT, _ = _gather_kernel(ids_cat, user_table.T, item_table.T)
    return (uT.T, iT.T)


# speedup vs baseline: 1.2160x; 1.2160x over previous
import functools

import jax
import jax.numpy as jnp
from jax import lax
from jax.experimental import pallas as pl
from jax.experimental.pallas import tpu as pltpu
from jax.experimental.pallas import tpu_sc as plsc

BATCH = 16384
EMBED_DIM = 64
LANES = 16

_NC = 2
_NS = 16
_NTC = 7813
_TPW = 489
_NRING = 3
_NGRP = (_TPW + _NRING - 1) // _NRING
_WRING = 8
_ACH = 2048
_CPW = BATCH // _NS
_TSUB = 256


def _extract_scalar(ref, pos):
    return ref[pl.ds(pos, LANES)][0]


@functools.partial(
    pl.kernel,
    mesh=plsc.VectorSubcoreMesh(core_axis_name="c", subcore_axis_name="s"),
    out_type=(
        jax.ShapeDtypeStruct((EMBED_DIM, BATCH), jnp.float32),
        jax.ShapeDtypeStruct((EMBED_DIM, BATCH), jnp.float32),
        jax.ShapeDtypeStruct((_NC * BATCH * EMBED_DIM,), jnp.float32),
    ),
    scratch_types=[
        pltpu.VMEM((_ACH,), jnp.int32),
        pltpu.VMEM((BATCH + LANES,), jnp.int32),
        pltpu.VMEM((BATCH + LANES,), jnp.int32),
        pltpu.VMEM((BATCH + LANES,), jnp.int32),
        pltpu.VMEM((BATCH + LANES,), jnp.int32),
        pltpu.VMEM((512 + LANES,), jnp.int32),
        pltpu.VMEM((512 + LANES,), jnp.int32),
        pltpu.VMEM((_NRING, EMBED_DIM, 128), jnp.float32),
        pltpu.VMEM((_WRING * EMBED_DIM,), jnp.float32),
        pltpu.VMEM((_TSUB * EMBED_DIM,), jnp.float32),
        pltpu.VMEM((EMBED_DIM, _TSUB), jnp.float32),
        [pltpu.SemaphoreType.DMA] * _NRING,
        pltpu.SemaphoreType.DMA,
    ],
    compiler_params=pltpu.CompilerParams(needs_layout_passes=False),
)
def _gather_kernel(ids_cat, utabT, itabT, uoutT, ioutT, exch,
                   allids, clist_id, clist_pos, sort_id, sort_pos,
                   begin_v, end_v, stage, tmpc, tbuf, colsT, sems, sem_w):
    core = lax.axis_index("c")
    sid = lax.axis_index("s")
    ibase = core * BATCH
    obase = core * BATCH * EMBED_DIM
    iota = lax.iota(jnp.int32, LANES)
    ones = jnp.ones((LANES,), jnp.int32)

    zeros = jnp.zeros((LANES,), jnp.int32)

    def zero_body(b, acc):
        begin_v[pl.ds(b * LANES, LANES)] = zeros
        return acc

    lax.fori_loop(0, 512 // LANES, zero_body, jnp.int32(0), unroll=False)

    def macro_body(a, pos):
        pltpu.sync_copy(ids_cat.at[pl.ds(ibase + a * _ACH, _ACH)], allids)

        def scan_body(k, pos):
            v = allids[pl.ds(k * LANES, LANES)]
            m = ((v >> 7) & (_NS - 1)) == sid
            plsc.store_compressed(clist_id.at[pl.ds(pos, LANES)], v, mask=m)
            plsc.store_compressed(clist_pos.at[pl.ds(pos, LANES)],
                                  iota + (a * _ACH + k * LANES), mask=m)
            plsc.addupdate_scatter(begin_v, [v >> 11], ones, mask=m)
            return pos + plsc.all_reduce_population_count(m)[0]

        return lax.fori_loop(0, _ACH // LANES, scan_body, pos, unroll=False)

    nkeep = lax.fori_loop(0, BATCH // _ACH, macro_body, jnp.int32(0),
                          unroll=False)

    def prefix_body(b, carry):
        v = begin_v[pl.ds(b * LANES, LANES)]
        s = plsc.cumsum(v) + carry
        end_v[pl.ds(b * LANES, LANES)] = s
        begin_v[pl.ds(b * LANES, LANES)] = s - v
        return s[LANES - 1]

    lax.fori_loop(0, 512 // LANES, prefix_body, jnp.int32(0), unroll=False)

    lane0 = iota == 0
    nchunk = (nkeep + LANES - 1) // LANES

    def place_body(k, acc):
        v = clist_id[pl.ds(k * LANES, LANES)]
        p = clist_pos[pl.ds(k * LANES, LANES)]
        for lane in range(LANES):
            @pl.when(k * LANES + lane < nkeep)
            def _place():
                idv = v[lane]
                pos = p[lane]
                slot = idv >> 11
                dst = _extract_scalar(begin_v, slot)
                plsc.store_scatter(sort_id,
                                   [jnp.full((LANES,), dst, jnp.int32)],
                                   jnp.full((LANES,), idv, jnp.int32),
                                   mask=lane0)
                plsc.store_scatter(sort_pos,
                                   [jnp.full((LANES,), dst, jnp.int32)],
                                   jnp.full((LANES,), pos, jnp.int32),
                                   mask=lane0)
                plsc.addupdate_scatter(begin_v,
                                       [jnp.full((LANES,), slot, jnp.int32)],
                                       ones, mask=lane0)
        return acc

    lax.fori_loop(0, nchunk, place_body, jnp.int32(0), unroll=False)

    def enqueue(t, q):
        tcg = sid + t * _NS
        off = pl.multiple_of(jnp.minimum(tcg, _NTC - 1) * 128, 128)
        inb = tcg < _NTC

        @pl.when(jnp.logical_and(inb, core == 0))
        def _u():
            pltpu.async_copy(utabT.at[:, pl.ds(off, 128)],
                             stage.at[q], sems[q])

        @pl.when(jnp.logical_and(inb, core == 1))
        def _i():
            pltpu.async_copy(itabT.at[:, pl.ds(off, 128)],
                             stage.at[q], sems[q])

    dummy_tc = utabT.at[:, pl.ds(0, 128)]
    dummy_row = exch.at[pl.ds(0, EMBED_DIM)]

    for q in range(_NRING):
        enqueue(q, q)

    def extract_tc(t, q, wcnt):
        bp = _extract_scalar(end_v, jnp.maximum(t - 1, 0))
        b0 = lax.select(t > 0, bp, jnp.int32(0))
        b1 = _extract_scalar(end_v, t)

        def elem_body(e, wcnt):
            idv = _extract_scalar(sort_id, e)
            pos = _extract_scalar(sort_pos, e)
            cvec = jnp.full((LANES,), idv & 127, jnp.int32)
            ws = wcnt & (_WRING - 1)

            @pl.when(wcnt >= _WRING)
            def _wring():
                pltpu.make_async_copy(
                    dummy_row, tmpc.at[pl.ds(ws * EMBED_DIM, EMBED_DIM)],
                    sem_w).wait()

            def gat_body(b, acc):
                dvec = iota + b * LANES
                vv = plsc.load_gather(stage.at[q], [dvec, cvec])
                tmpc[pl.ds(ws * EMBED_DIM + b * LANES, LANES)] = vv
                return acc

            lax.fori_loop(0, EMBED_DIM // LANES, gat_body, jnp.int32(0),
                          unroll=False)
            pltpu.async_copy(tmpc.at[pl.ds(ws * EMBED_DIM, EMBED_DIM)],
                             exch.at[pl.ds(obase + pos * EMBED_DIM,
                                           EMBED_DIM)],
                             sem_w)
            return wcnt + 1

        return lax.fori_loop(b0, b1, elem_body, wcnt, unroll=False)

    def group_body(g, wcnt):
        for q in range(_NRING):
            t = g * _NRING + q

            @pl.when(t * _NS + sid < _NTC)
            def _wait():
                pltpu.make_async_copy(dummy_tc, stage.at[q], sems[q]).wait()

            wcnt = lax.cond(t < _TPW,
                            lambda w: extract_tc(t, q, w),
                            lambda w: w, wcnt)
            enqueue(t + _NRING, q)
        return wcnt

    wcnt = lax.fori_loop(0, _NGRP, group_body, jnp.int32(0), unroll=False)

    def drain_body(d, acc):
        @pl.when(d < wcnt)
        def _drain():
            pltpu.make_async_copy(dummy_row, tmpc.at[pl.ds(0, EMBED_DIM)],
                                  sem_w).wait()
        return acc

    lax.fori_loop(0, _WRING, drain_body, jnp.int32(0), unroll=False)

    plsc.subcore_barrier()
    cbase = sid * _CPW

    def sub_body(sb, acc):
        coff = pl.multiple_of(cbase + sb * _TSUB, 128)
        pltpu.sync_copy(
            exch.at[pl.ds(obase + coff * EMBED_DIM, _TSUB * EMBED_DIM)],
            tbuf)

        def tr_d(d, acc):
            dsp = jnp.full((LANES,), d, jnp.int32)

            def tr_b(b, acc):
                cv = iota + b * LANES
                v = plsc.load_gather(tbuf, [cv * EMBED_DIM + d])
                plsc.store_scatter(colsT, [dsp, cv], v)
                return acc

            return lax.fori_loop(0, _TSUB // LANES, tr_b, acc,
                                 unroll=False)

        lax.fori_loop(0, EMBED_DIM, tr_d, jnp.int32(0), unroll=False)
        csl = pl.ds(coff, _TSUB)

        @pl.when(core == 0)
        def _wu():
            pltpu.sync_copy(colsT, uoutT.at[:, csl])

        @pl.when(core == 1)
        def _wi():
            pltpu.sync_copy(colsT, ioutT.at[:, csl])
        return acc

    lax.fori_loop(0, _CPW // _TSUB, sub_body, jnp.int32(0), unroll=False)


def kernel(user_ids, item_ids, user_table, item_table):
    ids_cat = jnp.concatenate([user_ids, item_ids])
    uT, iT, _ = _gather_kernel(ids_cat, user_table.T, item_table.T)
    return (uT.T, iT.T)


# speedup vs baseline: 1.3223x; 1.0875x over previous
import functools

import jax
import jax.numpy as jnp
from jax import lax
from jax.experimental import pallas as pl
from jax.experimental.pallas import tpu as pltpu
from jax.experimental.pallas import tpu_sc as plsc

BATCH = 16384
EMBED_DIM = 64
LANES = 16

_NC = 2
_NS = 16
_NTC = 7813
_TPW = 489
_NRING = 4
_NGRP = (_TPW + _NRING - 1) // _NRING
_WRING = 8
_ACH = 2048
_CPW = BATCH // _NS
_TSUB = 128


def _extract_scalar(ref, pos):
    return ref[pl.ds(pos, LANES)][0]


@functools.partial(
    pl.kernel,
    mesh=plsc.VectorSubcoreMesh(core_axis_name="c", subcore_axis_name="s"),
    out_type=(
        jax.ShapeDtypeStruct((EMBED_DIM, BATCH), jnp.float32),
        jax.ShapeDtypeStruct((EMBED_DIM, BATCH), jnp.float32),
        jax.ShapeDtypeStruct((_NC * BATCH * EMBED_DIM,), jnp.float32),
    ),
    scratch_types=[
        pltpu.VMEM((_ACH,), jnp.int32),
        pltpu.VMEM((BATCH + LANES,), jnp.int32),
        pltpu.VMEM((BATCH + LANES,), jnp.int32),
        pltpu.VMEM((BATCH + LANES,), jnp.int32),
        pltpu.VMEM((BATCH + LANES,), jnp.int32),
        pltpu.VMEM((512 + LANES,), jnp.int32),
        pltpu.VMEM((512 + LANES,), jnp.int32),
        pltpu.VMEM((_NRING, EMBED_DIM, 128), jnp.float32),
        pltpu.VMEM((_WRING * EMBED_DIM,), jnp.float32),
        pltpu.VMEM((_TSUB * EMBED_DIM,), jnp.float32),
        pltpu.VMEM((EMBED_DIM, _TSUB), jnp.float32),
        [pltpu.SemaphoreType.DMA] * _NRING,
        pltpu.SemaphoreType.DMA,
    ],
    compiler_params=pltpu.CompilerParams(needs_layout_passes=False),
)
def _gather_kernel(ids_cat, utabT, itabT, uoutT, ioutT, exch,
                   allids, clist_id, clist_pos, sort_id, sort_pos,
                   begin_v, end_v, stage, tmpc, tbuf, colsT, sems, sem_w):
    core = lax.axis_index("c")
    sid = lax.axis_index("s")
    ibase = core * BATCH
    obase = core * BATCH * EMBED_DIM
    iota = lax.iota(jnp.int32, LANES)
    ones = jnp.ones((LANES,), jnp.int32)

    zeros = jnp.zeros((LANES,), jnp.int32)

    def zero_body(b, acc):
        begin_v[pl.ds(b * LANES, LANES)] = zeros
        return acc

    lax.fori_loop(0, 512 // LANES, zero_body, jnp.int32(0), unroll=False)

    def macro_body(a, pos):
        pltpu.sync_copy(ids_cat.at[pl.ds(ibase + a * _ACH, _ACH)], allids)

        def scan_body(k, pos):
            v = allids[pl.ds(k * LANES, LANES)]
            m = ((v >> 7) & (_NS - 1)) == sid
            plsc.store_compressed(clist_id.at[pl.ds(pos, LANES)], v, mask=m)
            plsc.store_compressed(clist_pos.at[pl.ds(pos, LANES)],
                                  iota + (a * _ACH + k * LANES), mask=m)
            plsc.addupdate_scatter(begin_v, [v >> 11], ones, mask=m)
            return pos + plsc.all_reduce_population_count(m)[0]

        return lax.fori_loop(0, _ACH // LANES, scan_body, pos, unroll=False)

    nkeep = lax.fori_loop(0, BATCH // _ACH, macro_body, jnp.int32(0),
                          unroll=False)

    def prefix_body(b, carry):
        v = begin_v[pl.ds(b * LANES, LANES)]
        s = plsc.cumsum(v) + carry
        end_v[pl.ds(b * LANES, LANES)] = s
        begin_v[pl.ds(b * LANES, LANES)] = s - v
        return s[LANES - 1]

    lax.fori_loop(0, 512 // LANES, prefix_body, jnp.int32(0), unroll=False)

    lane0 = iota == 0
    nchunk = (nkeep + LANES - 1) // LANES

    def place_body(k, acc):
        v = clist_id[pl.ds(k * LANES, LANES)]
        p = clist_pos[pl.ds(k * LANES, LANES)]
        for lane in range(LANES):
            @pl.when(k * LANES + lane < nkeep)
            def _place():
                idv = v[lane]
                pos = p[lane]
                slot = idv >> 11
                dst = _extract_scalar(begin_v, slot)
                plsc.store_scatter(sort_id,
                                   [jnp.full((LANES,), dst, jnp.int32)],
                                   jnp.full((LANES,), idv, jnp.int32),
                                   mask=lane0)
                plsc.store_scatter(sort_pos,
                                   [jnp.full((LANES,), dst, jnp.int32)],
                                   jnp.full((LANES,), pos, jnp.int32),
                                   mask=lane0)
                plsc.addupdate_scatter(begin_v,
                                       [jnp.full((LANES,), slot, jnp.int32)],
                                       ones, mask=lane0)
        return acc

    lax.fori_loop(0, nchunk, place_body, jnp.int32(0), unroll=False)

    def enqueue(t, q):
        tcg = sid + t * _NS
        off = pl.multiple_of(jnp.minimum(tcg, _NTC - 1) * 128, 128)
        inb = tcg < _NTC

        @pl.when(jnp.logical_and(inb, core == 0))
        def _u():
            pltpu.async_copy(utabT.at[:, pl.ds(off, 128)],
                             stage.at[q], sems[q])

        @pl.when(jnp.logical_and(inb, core == 1))
        def _i():
            pltpu.async_copy(itabT.at[:, pl.ds(off, 128)],
                             stage.at[q], sems[q])

    dummy_tc = utabT.at[:, pl.ds(0, 128)]
    dummy_row = exch.at[pl.ds(0, EMBED_DIM)]

    for q in range(_NRING):
        enqueue(q, q)

    def extract_tc(t, q, wcnt):
        bp = _extract_scalar(end_v, jnp.maximum(t - 1, 0))
        b0 = lax.select(t > 0, bp, jnp.int32(0))
        b1 = _extract_scalar(end_v, t)

        def elem_body(e, wcnt):
            idv = _extract_scalar(sort_id, e)
            pos = _extract_scalar(sort_pos, e)
            cvec = jnp.full((LANES,), idv & 127, jnp.int32)
            ws = wcnt & (_WRING - 1)

            @pl.when(wcnt >= _WRING)
            def _wring():
                pltpu.make_async_copy(
                    dummy_row, tmpc.at[pl.ds(ws * EMBED_DIM, EMBED_DIM)],
                    sem_w).wait()

            def gat_body(b, acc):
                dvec = iota + b * LANES
                vv = plsc.load_gather(stage.at[q], [dvec, cvec])
                tmpc[pl.ds(ws * EMBED_DIM + b * LANES, LANES)] = vv
                return acc

            lax.fori_loop(0, EMBED_DIM // LANES, gat_body, jnp.int32(0),
                          unroll=False)
            pltpu.async_copy(tmpc.at[pl.ds(ws * EMBED_DIM, EMBED_DIM)],
                             exch.at[pl.ds(obase + pos * EMBED_DIM,
                                           EMBED_DIM)],
                             sem_w)
            return wcnt + 1

        return lax.fori_loop(b0, b1, elem_body, wcnt, unroll=False)

    def group_body(g, wcnt):
        for q in range(_NRING):
            t = g * _NRING + q

            @pl.when(t * _NS + sid < _NTC)
            def _wait():
                pltpu.make_async_copy(dummy_tc, stage.at[q], sems[q]).wait()

            wcnt = lax.cond(t < _TPW,
                            lambda w: extract_tc(t, q, w),
                            lambda w: w, wcnt)
            enqueue(t + _NRING, q)
        return wcnt

    wcnt = lax.fori_loop(0, _NGRP, group_body, jnp.int32(0), unroll=False)

    def drain_body(d, acc):
        @pl.when(d < wcnt)
        def _drain():
            pltpu.make_async_copy(dummy_row, tmpc.at[pl.ds(0, EMBED_DIM)],
                                  sem_w).wait()
        return acc

    lax.fori_loop(0, _WRING, drain_body, jnp.int32(0), unroll=False)

    plsc.subcore_barrier()
    cbase = sid * _CPW

    def sub_body(sb, acc):
        coff = pl.multiple_of(cbase + sb * _TSUB, 128)
        pltpu.sync_copy(
            exch.at[pl.ds(obase + coff * EMBED_DIM, _TSUB * EMBED_DIM)],
            tbuf)

        def tr_d(d, acc):
            dsp = jnp.full((LANES,), d, jnp.int32)

            def tr_b(b, acc):
                cv = iota + b * LANES
                v = plsc.load_gather(tbuf, [cv * EMBED_DIM + d])
                plsc.store_scatter(colsT, [dsp, cv], v)
                return acc

            return lax.fori_loop(0, _TSUB // LANES, tr_b, acc,
                                 unroll=False)

        lax.fori_loop(0, EMBED_DIM, tr_d, jnp.int32(0), unroll=False)
        csl = pl.ds(coff, _TSUB)

        @pl.when(core == 0)
        def _wu():
            pltpu.sync_copy(colsT, uoutT.at[:, csl])

        @pl.when(core == 1)
        def _wi():
            pltpu.sync_copy(colsT, ioutT.at[:, csl])
        return acc

    lax.fori_loop(0, _CPW // _TSUB, sub_body, jnp.int32(0), unroll=False)


def kernel(user_ids, item_ids, user_table, item_table):
    ids_cat = jnp.concatenate([user_ids, item_ids])
    uT, iT, _ = _gather_kernel(ids_cat, user_table.T, item_table.T)
    return (uT.T, iT.T)


# speedup vs baseline: 1.3925x; 1.0531x over previous
import functools

import jax
import jax.numpy as jnp
from jax import lax
from jax.experimental import pallas as pl
from jax.experimental.pallas import tpu as pltpu
from jax.experimental.pallas import tpu_sc as plsc

BATCH = 16384
EMBED_DIM = 64
LANES = 16

_NC = 2
_NS = 16
_NTC = 7813
_TPW = 489
_NRING = 5
_NGRP = (_TPW + _NRING - 1) // _NRING
_WRING = 8
_ACH = 2048
_CPW = BATCH // _NS
_TSUB = 128


def _extract_scalar(ref, pos):
    return ref[pl.ds(pos, LANES)][0]


@functools.partial(
    pl.kernel,
    mesh=plsc.VectorSubcoreMesh(core_axis_name="c", subcore_axis_name="s"),
    out_type=(
        jax.ShapeDtypeStruct((EMBED_DIM, BATCH), jnp.float32),
        jax.ShapeDtypeStruct((EMBED_DIM, BATCH), jnp.float32),
        jax.ShapeDtypeStruct((_NC * BATCH * EMBED_DIM,), jnp.float32),
    ),
    scratch_types=[
        pltpu.VMEM((_ACH,), jnp.int32),
        pltpu.VMEM((BATCH + LANES,), jnp.int32),
        pltpu.VMEM((BATCH + LANES,), jnp.int32),
        pltpu.VMEM((BATCH + LANES,), jnp.int32),
        pltpu.VMEM((BATCH + LANES,), jnp.int32),
        pltpu.VMEM((512 + LANES,), jnp.int32),
        pltpu.VMEM((512 + LANES,), jnp.int32),
        pltpu.VMEM((_NRING, EMBED_DIM, 128), jnp.float32),
        pltpu.VMEM((_WRING * EMBED_DIM,), jnp.float32),
        pltpu.VMEM((_TSUB * EMBED_DIM,), jnp.float32),
        pltpu.VMEM((EMBED_DIM, _TSUB), jnp.float32),
        [pltpu.SemaphoreType.DMA] * _NRING,
        pltpu.SemaphoreType.DMA,
    ],
    compiler_params=pltpu.CompilerParams(needs_layout_passes=False),
)
def _gather_kernel(ids_cat, utabT, itabT, uoutT, ioutT, exch,
                   allids, clist_id, clist_pos, sort_id, sort_pos,
                   begin_v, end_v, stage, tmpc, tbuf, colsT, sems, sem_w):
    core = lax.axis_index("c")
    sid = lax.axis_index("s")
    ibase = core * BATCH
    obase = core * BATCH * EMBED_DIM
    iota = lax.iota(jnp.int32, LANES)
    ones = jnp.ones((LANES,), jnp.int32)

    zeros = jnp.zeros((LANES,), jnp.int32)

    def zero_body(b, acc):
        begin_v[pl.ds(b * LANES, LANES)] = zeros
        return acc

    lax.fori_loop(0, 512 // LANES, zero_body, jnp.int32(0), unroll=False)

    def macro_body(a, pos):
        pltpu.sync_copy(ids_cat.at[pl.ds(ibase + a * _ACH, _ACH)], allids)

        def scan_body(k, pos):
            v = allids[pl.ds(k * LANES, LANES)]
            m = ((v >> 7) & (_NS - 1)) == sid
            plsc.store_compressed(clist_id.at[pl.ds(pos, LANES)], v, mask=m)
            plsc.store_compressed(clist_pos.at[pl.ds(pos, LANES)],
                                  iota + (a * _ACH + k * LANES), mask=m)
            plsc.addupdate_scatter(begin_v, [v >> 11], ones, mask=m)
            return pos + plsc.all_reduce_population_count(m)[0]

        return lax.fori_loop(0, _ACH // LANES, scan_body, pos, unroll=False)

    nkeep = lax.fori_loop(0, BATCH // _ACH, macro_body, jnp.int32(0),
                          unroll=False)

    def prefix_body(b, carry):
        v = begin_v[pl.ds(b * LANES, LANES)]
        s = plsc.cumsum(v) + carry
        end_v[pl.ds(b * LANES, LANES)] = s
        begin_v[pl.ds(b * LANES, LANES)] = s - v
        return s[LANES - 1]

    lax.fori_loop(0, 512 // LANES, prefix_body, jnp.int32(0), unroll=False)

    lane0 = iota == 0
    nchunk = (nkeep + LANES - 1) // LANES

    def place_body(k, acc):
        v = clist_id[pl.ds(k * LANES, LANES)]
        p = clist_pos[pl.ds(k * LANES, LANES)]
        for lane in range(LANES):
            @pl.when(k * LANES + lane < nkeep)
            def _place():
                idv = v[lane]
                pos = p[lane]
                slot = idv >> 11
                dst = _extract_scalar(begin_v, slot)
                plsc.store_scatter(sort_id,
                                   [jnp.full((LANES,), dst, jnp.int32)],
                                   jnp.full((LANES,), idv, jnp.int32),
                                   mask=lane0)
                plsc.store_scatter(sort_pos,
                                   [jnp.full((LANES,), dst, jnp.int32)],
                                   jnp.full((LANES,), pos, jnp.int32),
                                   mask=lane0)
                plsc.addupdate_scatter(begin_v,
                                       [jnp.full((LANES,), slot, jnp.int32)],
                                       ones, mask=lane0)
        return acc

    lax.fori_loop(0, nchunk, place_body, jnp.int32(0), unroll=False)

    def enqueue(t, q):
        tcg = sid + t * _NS
        off = pl.multiple_of(jnp.minimum(tcg, _NTC - 1) * 128, 128)
        inb = tcg < _NTC

        @pl.when(jnp.logical_and(inb, core == 0))
        def _u():
            pltpu.async_copy(utabT.at[:, pl.ds(off, 128)],
                             stage.at[q], sems[q])

        @pl.when(jnp.logical_and(inb, core == 1))
        def _i():
            pltpu.async_copy(itabT.at[:, pl.ds(off, 128)],
                             stage.at[q], sems[q])

    dummy_tc = utabT.at[:, pl.ds(0, 128)]
    dummy_row = exch.at[pl.ds(0, EMBED_DIM)]

    for q in range(_NRING):
        enqueue(q, q)

    def extract_tc(t, q, wcnt):
        bp = _extract_scalar(end_v, jnp.maximum(t - 1, 0))
        b0 = lax.select(t > 0, bp, jnp.int32(0))
        b1 = _extract_scalar(end_v, t)

        def elem_body(e, wcnt):
            idv = _extract_scalar(sort_id, e)
            pos = _extract_scalar(sort_pos, e)
            cvec = jnp.full((LANES,), idv & 127, jnp.int32)
            ws = wcnt & (_WRING - 1)

            @pl.when(wcnt >= _WRING)
            def _wring():
                pltpu.make_async_copy(
                    dummy_row, tmpc.at[pl.ds(ws * EMBED_DIM, EMBED_DIM)],
                    sem_w).wait()

            def gat_body(b, acc):
                dvec = iota + b * LANES
                vv = plsc.load_gather(stage.at[q], [dvec, cvec])
                tmpc[pl.ds(ws * EMBED_DIM + b * LANES, LANES)] = vv
                return acc

            lax.fori_loop(0, EMBED_DIM // LANES, gat_body, jnp.int32(0),
                          unroll=False)
            pltpu.async_copy(tmpc.at[pl.ds(ws * EMBED_DIM, EMBED_DIM)],
                             exch.at[pl.ds(obase + pos * EMBED_DIM,
                                           EMBED_DIM)],
                             sem_w)
            return wcnt + 1

        return lax.fori_loop(b0, b1, elem_body, wcnt, unroll=False)

    def group_body(g, wcnt):
        for q in range(_NRING):
            t = g * _NRING + q

            @pl.when(t * _NS + sid < _NTC)
            def _wait():
                pltpu.make_async_copy(dummy_tc, stage.at[q], sems[q]).wait()

            wcnt = lax.cond(t < _TPW,
                            lambda w: extract_tc(t, q, w),
                            lambda w: w, wcnt)
            enqueue(t + _NRING, q)
        return wcnt

    wcnt = lax.fori_loop(0, _NGRP, group_body, jnp.int32(0), unroll=False)

    def drain_body(d, acc):
        @pl.when(d < wcnt)
        def _drain():
            pltpu.make_async_copy(dummy_row, tmpc.at[pl.ds(0, EMBED_DIM)],
                                  sem_w).wait()
        return acc

    lax.fori_loop(0, _WRING, drain_body, jnp.int32(0), unroll=False)

    plsc.subcore_barrier()
    cbase = sid * _CPW

    def sub_body(sb, acc):
        coff = pl.multiple_of(cbase + sb * _TSUB, 128)
        pltpu.sync_copy(
            exch.at[pl.ds(obase + coff * EMBED_DIM, _TSUB * EMBED_DIM)],
            tbuf)

        def tr_d(d, acc):
            dsp = jnp.full((LANES,), d, jnp.int32)

            def tr_b(b, acc):
                cv = iota + b * LANES
                v = plsc.load_gather(tbuf, [cv * EMBED_DIM + d])
                plsc.store_scatter(colsT, [dsp, cv], v)
                return acc

            return lax.fori_loop(0, _TSUB // LANES, tr_b, acc,
                                 unroll=False)

        lax.fori_loop(0, EMBED_DIM, tr_d, jnp.int32(0), unroll=False)
        csl = pl.ds(coff, _TSUB)

        @pl.when(core == 0)
        def _wu():
            pltpu.sync_copy(colsT, uoutT.at[:, csl])

        @pl.when(core == 1)
        def _wi():
            pltpu.sync_copy(colsT, ioutT.at[:, csl])
        return acc

    lax.fori_loop(0, _CPW // _TSUB, sub_body, jnp.int32(0), unroll=False)


def kernel(user_ids, item_ids, user_table, item_table):
    ids_cat = jnp.concatenate([user_ids, item_ids])
    uT, iT, _ = _gather_kernel(ids_cat, user_table.T, item_table.T)
    return (uT.T, iT.T)


# speedup vs baseline: 1.4688x; 1.0548x over previous
import functools

import jax
import jax.numpy as jnp
from jax import lax
from jax.experimental import pallas as pl
from jax.experimental.pallas import tpu as pltpu
from jax.experimental.pallas import tpu_sc as plsc

BATCH = 16384
EMBED_DIM = 64
LANES = 16

_NC = 2
_NS = 16
_NTC = 7813
_TPW = 489
_NRING = 8
_NGRP = (_TPW + _NRING - 1) // _NRING
_WRING = 8
_ACH = 2048
_CPW = BATCH // _NS
_TSUB = 128


def _extract_scalar(ref, pos):
    return ref[pl.ds(pos, LANES)][0]


@functools.partial(
    pl.kernel,
    mesh=plsc.VectorSubcoreMesh(core_axis_name="c", subcore_axis_name="s"),
    out_type=(
        jax.ShapeDtypeStruct((EMBED_DIM, BATCH), jnp.float32),
        jax.ShapeDtypeStruct((EMBED_DIM, BATCH), jnp.float32),
        jax.ShapeDtypeStruct((_NC * BATCH * EMBED_DIM,), jnp.float32),
    ),
    scratch_types=[
        pltpu.VMEM((_ACH,), jnp.int32),
        pltpu.VMEM((BATCH + LANES,), jnp.int32),
        pltpu.VMEM((BATCH + LANES,), jnp.int32),
        pltpu.VMEM((512 + LANES,), jnp.int32),
        pltpu.VMEM((512 + LANES,), jnp.int32),
        pltpu.VMEM((_NRING, EMBED_DIM, 128), jnp.float32),
        pltpu.VMEM((_WRING * EMBED_DIM,), jnp.float32),
        pltpu.VMEM((_TSUB * EMBED_DIM,), jnp.float32),
        pltpu.VMEM((EMBED_DIM, _TSUB), jnp.float32),
        [pltpu.SemaphoreType.DMA] * _NRING,
        pltpu.SemaphoreType.DMA,
    ],
    compiler_params=pltpu.CompilerParams(needs_layout_passes=False),
)
def _gather_kernel(ids_cat, utabT, itabT, uoutT, ioutT, exch,
                   allids, clist_id, sort_id,
                   begin_v, end_v, stage, tmpc, tbuf, colsT, sems, sem_w):
    core = lax.axis_index("c")
    sid = lax.axis_index("s")
    ibase = core * BATCH
    obase = core * BATCH * EMBED_DIM
    iota = lax.iota(jnp.int32, LANES)
    ones = jnp.ones((LANES,), jnp.int32)

    zeros = jnp.zeros((LANES,), jnp.int32)

    def zero_body(b, acc):
        begin_v[pl.ds(b * LANES, LANES)] = zeros
        return acc

    lax.fori_loop(0, 512 // LANES, zero_body, jnp.int32(0), unroll=False)

    def macro_body(a, pos):
        pltpu.sync_copy(ids_cat.at[pl.ds(ibase + a * _ACH, _ACH)], allids)

        def scan_body(k, pos):
            v = allids[pl.ds(k * LANES, LANES)]
            m = ((v >> 7) & (_NS - 1)) == sid
            e = (((v >> 11) << 21) | ((v & 127) << 14)
                 | (iota + (a * _ACH + k * LANES)))
            plsc.store_compressed(clist_id.at[pl.ds(pos, LANES)], e, mask=m)
            plsc.addupdate_scatter(begin_v, [v >> 11], ones, mask=m)
            return pos + plsc.all_reduce_population_count(m)[0]

        return lax.fori_loop(0, _ACH // LANES, scan_body, pos, unroll=False)

    nkeep = lax.fori_loop(0, BATCH // _ACH, macro_body, jnp.int32(0),
                          unroll=False)

    def prefix_body(b, carry):
        v = begin_v[pl.ds(b * LANES, LANES)]
        s = plsc.cumsum(v) + carry
        end_v[pl.ds(b * LANES, LANES)] = s
        begin_v[pl.ds(b * LANES, LANES)] = s - v
        return s[LANES - 1]

    lax.fori_loop(0, 512 // LANES, prefix_body, jnp.int32(0), unroll=False)

    lane0 = iota == 0
    nchunk = (nkeep + LANES - 1) // LANES

    def place_body(k, acc):
        v = clist_id[pl.ds(k * LANES, LANES)]
        for lane in range(LANES):
            @pl.when(k * LANES + lane < nkeep)
            def _place():
                ev = v[lane]
                slot = ev >> 21
                dst = _extract_scalar(begin_v, slot)
                plsc.store_scatter(sort_id,
                                   [jnp.full((LANES,), dst, jnp.int32)],
                                   jnp.full((LANES,), ev & 0x1FFFFF,
                                            jnp.int32),
                                   mask=lane0)
                plsc.addupdate_scatter(begin_v,
                                       [jnp.full((LANES,), slot, jnp.int32)],
                                       ones, mask=lane0)
        return acc

    lax.fori_loop(0, nchunk, place_body, jnp.int32(0), unroll=False)

    def enqueue(t, q):
        tcg = sid + t * _NS
        off = pl.multiple_of(jnp.minimum(tcg, _NTC - 1) * 128, 128)
        inb = tcg < _NTC

        @pl.when(jnp.logical_and(inb, core == 0))
        def _u():
            pltpu.async_copy(utabT.at[:, pl.ds(off, 128)],
                             stage.at[q], sems[q])

        @pl.when(jnp.logical_and(inb, core == 1))
        def _i():
            pltpu.async_copy(itabT.at[:, pl.ds(off, 128)],
                             stage.at[q], sems[q])

    dummy_tc = utabT.at[:, pl.ds(0, 128)]
    dummy_row = exch.at[pl.ds(0, EMBED_DIM)]

    for q in range(_NRING):
        enqueue(q, q)

    def extract_tc(t, q, wcnt):
        bp = _extract_scalar(end_v, jnp.maximum(t - 1, 0))
        b0 = lax.select(t > 0, bp, jnp.int32(0))
        b1 = _extract_scalar(end_v, t)

        def elem_body(e, wcnt):
            ev = _extract_scalar(sort_id, e)
            pos = ev & 16383
            cvec = jnp.full((LANES,), ev >> 14, jnp.int32)
            ws = wcnt & (_WRING - 1)

            @pl.when(wcnt >= _WRING)
            def _wring():
                pltpu.make_async_copy(
                    dummy_row, tmpc.at[pl.ds(ws * EMBED_DIM, EMBED_DIM)],
                    sem_w).wait()

            def gat_body(b, acc):
                dvec = iota + b * LANES
                vv = plsc.load_gather(stage.at[q], [dvec, cvec])
                tmpc[pl.ds(ws * EMBED_DIM + b * LANES, LANES)] = vv
                return acc

            lax.fori_loop(0, EMBED_DIM // LANES, gat_body, jnp.int32(0),
                          unroll=False)
            pltpu.async_copy(tmpc.at[pl.ds(ws * EMBED_DIM, EMBED_DIM)],
                             exch.at[pl.ds(obase + pos * EMBED_DIM,
                                           EMBED_DIM)],
                             sem_w)
            return wcnt + 1

        return lax.fori_loop(b0, b1, elem_body, wcnt, unroll=False)

    def group_body(g, wcnt):
        for q in range(_NRING):
            t = g * _NRING + q

            @pl.when(t * _NS + sid < _NTC)
            def _wait():
                pltpu.make_async_copy(dummy_tc, stage.at[q], sems[q]).wait()

            wcnt = lax.cond(t < _TPW,
                            lambda w: extract_tc(t, q, w),
                            lambda w: w, wcnt)
            enqueue(t + _NRING, q)
        return wcnt

    wcnt = lax.fori_loop(0, _NGRP, group_body, jnp.int32(0), unroll=False)

    def drain_body(d, acc):
        @pl.when(d < wcnt)
        def _drain():
            pltpu.make_async_copy(dummy_row, tmpc.at[pl.ds(0, EMBED_DIM)],
                                  sem_w).wait()
        return acc

    lax.fori_loop(0, _WRING, drain_body, jnp.int32(0), unroll=False)

    plsc.subcore_barrier()
    cbase = sid * _CPW

    def sub_body(sb, acc):
        coff = pl.multiple_of(cbase + sb * _TSUB, 128)
        pltpu.sync_copy(
            exch.at[pl.ds(obase + coff * EMBED_DIM, _TSUB * EMBED_DIM)],
            tbuf)

        def tr_d(d, acc):
            dsp = jnp.full((LANES,), d, jnp.int32)

            def tr_b(b, acc):
                cv = iota + b * LANES
                v = plsc.load_gather(tbuf, [cv * EMBED_DIM + d])
                plsc.store_scatter(colsT, [dsp, cv], v)
                return acc

            return lax.fori_loop(0, _TSUB // LANES, tr_b, acc,
                                 unroll=False)

        lax.fori_loop(0, EMBED_DIM, tr_d, jnp.int32(0), unroll=False)
        csl = pl.ds(coff, _TSUB)

        @pl.when(core == 0)
        def _wu():
            pltpu.sync_copy(colsT, uoutT.at[:, csl])

        @pl.when(core == 1)
        def _wi():
            pltpu.sync_copy(colsT, ioutT.at[:, csl])
        return acc

    lax.fori_loop(0, _CPW // _TSUB, sub_body, jnp.int32(0), unroll=False)


def kernel(user_ids, item_ids, user_table, item_table):
    ids_cat = jnp.concatenate([user_ids, item_ids])
    uT, iT, _ = _gather_kernel(ids_cat, user_table.T, item_table.T)
    return (uT.T, iT.T)


# speedup vs baseline: 1.4867x; 1.0122x over previous
import functools

import jax
import jax.numpy as jnp
from jax import lax
from jax.experimental import pallas as pl
from jax.experimental.pallas import tpu as pltpu
from jax.experimental.pallas import tpu_sc as plsc

BATCH = 16384
EMBED_DIM = 64
LANES = 16

_NC = 2
_NS = 16
_NTC = 7813
_TPW = 489
_NRING = 9
_NGRP = (_TPW + _NRING - 1) // _NRING
_WRING = 16
_ACH = 2048
_CPW = BATCH // _NS
_TSUB = 128


def _extract_scalar(ref, pos):
    return ref[pl.ds(pos, LANES)][0]


@functools.partial(
    pl.kernel,
    mesh=plsc.VectorSubcoreMesh(core_axis_name="c", subcore_axis_name="s"),
    out_type=(
        jax.ShapeDtypeStruct((EMBED_DIM, BATCH), jnp.float32),
        jax.ShapeDtypeStruct((EMBED_DIM, BATCH), jnp.float32),
        jax.ShapeDtypeStruct((_NC * BATCH * EMBED_DIM,), jnp.float32),
    ),
    scratch_types=[
        pltpu.VMEM((_ACH,), jnp.int32),
        pltpu.VMEM((BATCH + LANES,), jnp.int32),
        pltpu.VMEM((BATCH + LANES,), jnp.int32),
        pltpu.VMEM((512 + LANES,), jnp.int32),
        pltpu.VMEM((512 + LANES,), jnp.int32),
        pltpu.VMEM((_NRING, EMBED_DIM, 128), jnp.float32),
        pltpu.VMEM((_WRING * EMBED_DIM,), jnp.float32),
        pltpu.VMEM((_TSUB * EMBED_DIM,), jnp.float32),
        pltpu.VMEM((EMBED_DIM, _TSUB), jnp.float32),
        [pltpu.SemaphoreType.DMA] * _NRING,
        pltpu.SemaphoreType.DMA,
    ],
    compiler_params=pltpu.CompilerParams(needs_layout_passes=False),
)
def _gather_kernel(ids_cat, utabT, itabT, uoutT, ioutT, exch,
                   allids, clist_id, sort_id,
                   begin_v, end_v, stage, tmpc, tbuf, colsT, sems, sem_w):
    core = lax.axis_index("c")
    sid = lax.axis_index("s")
    ibase = core * BATCH
    obase = core * BATCH * EMBED_DIM
    iota = lax.iota(jnp.int32, LANES)
    ones = jnp.ones((LANES,), jnp.int32)

    zeros = jnp.zeros((LANES,), jnp.int32)

    def zero_body(b, acc):
        begin_v[pl.ds(b * LANES, LANES)] = zeros
        return acc

    lax.fori_loop(0, 512 // LANES, zero_body, jnp.int32(0), unroll=False)

    def macro_body(a, pos):
        pltpu.sync_copy(ids_cat.at[pl.ds(ibase + a * _ACH, _ACH)], allids)

        def scan_body(k, pos):
            v = allids[pl.ds(k * LANES, LANES)]
            m = ((v >> 7) & (_NS - 1)) == sid
            e = (((v >> 11) << 21) | ((v & 127) << 14)
                 | (iota + (a * _ACH + k * LANES)))
            plsc.store_compressed(clist_id.at[pl.ds(pos, LANES)], e, mask=m)
            plsc.addupdate_scatter(begin_v, [v >> 11], ones, mask=m)
            return pos + plsc.all_reduce_population_count(m)[0]

        return lax.fori_loop(0, _ACH // LANES, scan_body, pos, unroll=False)

    nkeep = lax.fori_loop(0, BATCH // _ACH, macro_body, jnp.int32(0),
                          unroll=False)

    def prefix_body(b, carry):
        v = begin_v[pl.ds(b * LANES, LANES)]
        s = plsc.cumsum(v) + carry
        end_v[pl.ds(b * LANES, LANES)] = s
        begin_v[pl.ds(b * LANES, LANES)] = s - v
        return s[LANES - 1]

    lax.fori_loop(0, 512 // LANES, prefix_body, jnp.int32(0), unroll=False)

    lane0 = iota == 0
    nchunk = (nkeep + LANES - 1) // LANES

    def place_body(k, acc):
        v = clist_id[pl.ds(k * LANES, LANES)]
        for lane in range(LANES):
            @pl.when(k * LANES + lane < nkeep)
            def _place():
                ev = v[lane]
                slot = ev >> 21
                dst = _extract_scalar(begin_v, slot)
                plsc.store_scatter(sort_id,
                                   [jnp.full((LANES,), dst, jnp.int32)],
                                   jnp.full((LANES,), ev & 0x1FFFFF,
                                            jnp.int32),
                                   mask=lane0)
                plsc.addupdate_scatter(begin_v,
                                       [jnp.full((LANES,), slot, jnp.int32)],
                                       ones, mask=lane0)
        return acc

    lax.fori_loop(0, nchunk, place_body, jnp.int32(0), unroll=False)

    def enqueue(t, q):
        tcg = sid + t * _NS
        off = pl.multiple_of(jnp.minimum(tcg, _NTC - 1) * 128, 128)
        inb = tcg < _NTC

        @pl.when(jnp.logical_and(inb, core == 0))
        def _u():
            pltpu.async_copy(utabT.at[:, pl.ds(off, 128)],
                             stage.at[q], sems[q])

        @pl.when(jnp.logical_and(inb, core == 1))
        def _i():
            pltpu.async_copy(itabT.at[:, pl.ds(off, 128)],
                             stage.at[q], sems[q])

    dummy_tc = utabT.at[:, pl.ds(0, 128)]
    dummy_row = exch.at[pl.ds(0, EMBED_DIM)]

    for q in range(_NRING):
        enqueue(q, q)

    def extract_tc(t, q, wcnt):
        bp = _extract_scalar(end_v, jnp.maximum(t - 1, 0))
        b0 = lax.select(t > 0, bp, jnp.int32(0))
        b1 = _extract_scalar(end_v, t)

        def elem_body(e, wcnt):
            ev = _extract_scalar(sort_id, e)
            pos = ev & 16383
            cvec = jnp.full((LANES,), ev >> 14, jnp.int32)
            ws = wcnt & (_WRING - 1)

            @pl.when(wcnt >= _WRING)
            def _wring():
                pltpu.make_async_copy(
                    dummy_row, tmpc.at[pl.ds(ws * EMBED_DIM, EMBED_DIM)],
                    sem_w).wait()

            def gat_body(b, acc):
                dvec = iota + b * LANES
                vv = plsc.load_gather(stage.at[q], [dvec, cvec])
                tmpc[pl.ds(ws * EMBED_DIM + b * LANES, LANES)] = vv
                return acc

            lax.fori_loop(0, EMBED_DIM // LANES, gat_body, jnp.int32(0),
                          unroll=False)
            pltpu.async_copy(tmpc.at[pl.ds(ws * EMBED_DIM, EMBED_DIM)],
                             exch.at[pl.ds(obase + pos * EMBED_DIM,
                                           EMBED_DIM)],
                             sem_w)
            return wcnt + 1

        return lax.fori_loop(b0, b1, elem_body, wcnt, unroll=False)

    def group_body(g, wcnt):
        for q in range(_NRING):
            t = g * _NRING + q

            @pl.when(t * _NS + sid < _NTC)
            def _wait():
                pltpu.make_async_copy(dummy_tc, stage.at[q], sems[q]).wait()

            wcnt = lax.cond(t < _TPW,
                            lambda w: extract_tc(t, q, w),
                            lambda w: w, wcnt)
            enqueue(t + _NRING, q)
        return wcnt

    wcnt = lax.fori_loop(0, _NGRP, group_body, jnp.int32(0), unroll=False)

    def drain_body(d, acc):
        @pl.when(d < wcnt)
        def _drain():
            pltpu.make_async_copy(dummy_row, tmpc.at[pl.ds(0, EMBED_DIM)],
                                  sem_w).wait()
        return acc

    lax.fori_loop(0, _WRING, drain_body, jnp.int32(0), unroll=False)

    plsc.subcore_barrier()
    cbase = sid * _CPW

    def sub_body(sb, acc):
        coff = pl.multiple_of(cbase + sb * _TSUB, 128)
        pltpu.sync_copy(
            exch.at[pl.ds(obase + coff * EMBED_DIM, _TSUB * EMBED_DIM)],
            tbuf)

        def tr_d(d, acc):
            dsp = jnp.full((LANES,), d, jnp.int32)

            def tr_b(b, acc):
                cv = iota + b * LANES
                v = plsc.load_gather(tbuf, [cv * EMBED_DIM + d])
                plsc.store_scatter(colsT, [dsp, cv], v)
                return acc

            return lax.fori_loop(0, _TSUB // LANES, tr_b, acc,
                                 unroll=False)

        lax.fori_loop(0, EMBED_DIM, tr_d, jnp.int32(0), unroll=False)
        csl = pl.ds(coff, _TSUB)

        @pl.when(core == 0)
        def _wu():
            pltpu.sync_copy(colsT, uoutT.at[:, csl])

        @pl.when(core == 1)
        def _wi():
            pltpu.sync_copy(colsT, ioutT.at[:, csl])
        return acc

    lax.fori_loop(0, _CPW // _TSUB, sub_body, jnp.int32(0), unroll=False)


def kernel(user_ids, item_ids, user_table, item_table):
    ids_cat = jnp.concatenate([user_ids, item_ids])
    uT, iT, _ = _gather_kernel(ids_cat, user_table.T, item_table.T)
    return (uT.T, iT.T)


# speedup vs baseline: 1.4890x; 1.0015x over previous
import functools

import jax
import jax.numpy as jnp
from jax import lax
from jax.experimental import pallas as pl
from jax.experimental.pallas import tpu as pltpu
from jax.experimental.pallas import tpu_sc as plsc

BATCH = 16384
EMBED_DIM = 64
LANES = 16

_NC = 2
_NS = 16
_NTC = 7813
_TPW = 489
_NRING = 9
_NGRP = (_TPW + _NRING - 1) // _NRING
_WRING = 16
_ACH = 2048
_CPW = BATCH // _NS
_TSUB = 128


def _extract_scalar(ref, pos):
    return ref[pl.ds(pos, LANES)][0]


@functools.partial(
    pl.kernel,
    mesh=plsc.VectorSubcoreMesh(core_axis_name="c", subcore_axis_name="s"),
    out_type=(
        jax.ShapeDtypeStruct((EMBED_DIM, BATCH), jnp.float32),
        jax.ShapeDtypeStruct((EMBED_DIM, BATCH), jnp.float32),
        jax.ShapeDtypeStruct((_NC * BATCH * EMBED_DIM,), jnp.float32),
    ),
    scratch_types=[
        pltpu.VMEM((_ACH,), jnp.int32),
        pltpu.VMEM((BATCH + LANES,), jnp.int32),
        pltpu.VMEM((BATCH + LANES,), jnp.int32),
        pltpu.VMEM((512 + LANES,), jnp.int32),
        pltpu.VMEM((512 + LANES,), jnp.int32),
        pltpu.VMEM((_NRING, EMBED_DIM, 128), jnp.float32),
        pltpu.VMEM((_WRING * EMBED_DIM,), jnp.float32),
        pltpu.VMEM((_TSUB * EMBED_DIM,), jnp.float32),
        pltpu.VMEM((EMBED_DIM, _TSUB), jnp.float32),
        [pltpu.SemaphoreType.DMA] * _NRING,
        pltpu.SemaphoreType.DMA,
    ],
    compiler_params=pltpu.CompilerParams(needs_layout_passes=False),
)
def _gather_kernel(ids_cat, utabT, itabT, uoutT, ioutT, exch,
                   allids, clist_id, sort_id,
                   begin_v, end_v, stage, tmpc, tbuf, colsT, sems, sem_w):
    core = lax.axis_index("c")
    sid = lax.axis_index("s")
    ibase = core * BATCH
    obase = core * BATCH * EMBED_DIM
    iota = lax.iota(jnp.int32, LANES)
    ones = jnp.ones((LANES,), jnp.int32)

    zeros = jnp.zeros((LANES,), jnp.int32)

    def zero_body(b, acc):
        begin_v[pl.ds(b * LANES, LANES)] = zeros
        return acc

    lax.fori_loop(0, 512 // LANES, zero_body, jnp.int32(0), unroll=False)

    def macro_body(a, pos):
        pltpu.sync_copy(ids_cat.at[pl.ds(ibase + a * _ACH, _ACH)], allids)

        def scan_body(k, pos):
            v = allids[pl.ds(k * LANES, LANES)]
            m = ((v >> 7) & (_NS - 1)) == sid
            e = (((v >> 11) << 21) | ((v & 127) << 14)
                 | (iota + (a * _ACH + k * LANES)))
            plsc.store_compressed(clist_id.at[pl.ds(pos, LANES)], e, mask=m)
            plsc.addupdate_scatter(begin_v, [v >> 11], ones, mask=m)
            return pos + plsc.all_reduce_population_count(m)[0]

        return lax.fori_loop(0, _ACH // LANES, scan_body, pos, unroll=False)

    nkeep = lax.fori_loop(0, BATCH // _ACH, macro_body, jnp.int32(0),
                          unroll=False)

    def prefix_body(b, carry):
        v = begin_v[pl.ds(b * LANES, LANES)]
        s = plsc.cumsum(v) + carry
        end_v[pl.ds(b * LANES, LANES)] = s
        begin_v[pl.ds(b * LANES, LANES)] = s - v
        return s[LANES - 1]

    lax.fori_loop(0, 512 // LANES, prefix_body, jnp.int32(0), unroll=False)

    lane0 = iota == 0
    nchunk = (nkeep + LANES - 1) // LANES

    def place_body(k, acc):
        v = clist_id[pl.ds(k * LANES, LANES)]
        for lane in range(LANES):
            @pl.when(k * LANES + lane < nkeep)
            def _place():
                ev = v[lane]
                slot = ev >> 21
                dst = _extract_scalar(begin_v, slot)
                plsc.store_scatter(sort_id,
                                   [jnp.full((LANES,), dst, jnp.int32)],
                                   jnp.full((LANES,), ev & 0x1FFFFF,
                                            jnp.int32),
                                   mask=lane0)
                plsc.addupdate_scatter(begin_v,
                                       [jnp.full((LANES,), slot, jnp.int32)],
                                       ones, mask=lane0)
        return acc

    lax.fori_loop(0, nchunk, place_body, jnp.int32(0), unroll=False)

    def enqueue(t, q):
        tcg = sid + t * _NS
        off = pl.multiple_of(jnp.minimum(tcg, _NTC - 1) * 128, 128)
        inb = tcg < _NTC

        @pl.when(jnp.logical_and(inb, core == 0))
        def _u():
            pltpu.async_copy(utabT.at[:, pl.ds(off, 128)],
                             stage.at[q], sems[q])

        @pl.when(jnp.logical_and(inb, core == 1))
        def _i():
            pltpu.async_copy(itabT.at[:, pl.ds(off, 128)],
                             stage.at[q], sems[q])

    dummy_tc = utabT.at[:, pl.ds(0, 128)]
    dummy_row = exch.at[pl.ds(0, EMBED_DIM)]
    dummy_batch = exch.at[pl.ds(0, _WRING * EMBED_DIM)]

    for q in range(_NRING):
        enqueue(q, q)

    def extract_tc(t, q, wcnt):
        bp = _extract_scalar(end_v, jnp.maximum(t - 1, 0))
        b0 = lax.select(t > 0, bp, jnp.int32(0))
        b1 = _extract_scalar(end_v, t)

        def elem_body(e, wcnt):
            ev = _extract_scalar(sort_id, e)
            pos = ev & 16383
            cvec = jnp.full((LANES,), ev >> 14, jnp.int32)
            ws = wcnt & (_WRING - 1)

            @pl.when(jnp.logical_and(wcnt >= _WRING, ws == 0))
            def _wring():
                pltpu.make_async_copy(dummy_batch, tmpc, sem_w).wait()

            def gat_body(b, acc):
                dvec = iota + b * LANES
                vv = plsc.load_gather(stage.at[q], [dvec, cvec])
                tmpc[pl.ds(ws * EMBED_DIM + b * LANES, LANES)] = vv
                return acc

            lax.fori_loop(0, EMBED_DIM // LANES, gat_body, jnp.int32(0),
                          unroll=False)
            pltpu.async_copy(tmpc.at[pl.ds(ws * EMBED_DIM, EMBED_DIM)],
                             exch.at[pl.ds(obase + pos * EMBED_DIM,
                                           EMBED_DIM)],
                             sem_w)
            return wcnt + 1

        return lax.fori_loop(b0, b1, elem_body, wcnt, unroll=False)

    def group_body(g, wcnt):
        for q in range(_NRING):
            t = g * _NRING + q

            @pl.when(t * _NS + sid < _NTC)
            def _wait():
                pltpu.make_async_copy(dummy_tc, stage.at[q], sems[q]).wait()

            wcnt = lax.cond(t < _TPW,
                            lambda w: extract_tc(t, q, w),
                            lambda w: w, wcnt)
            enqueue(t + _NRING, q)
        return wcnt

    wcnt = lax.fori_loop(0, _NGRP, group_body, jnp.int32(0), unroll=False)

    ndrain = lax.select(wcnt >= _WRING,
                        ((wcnt - 1) & (_WRING - 1)) + 1, wcnt)

    def drain_body(d, acc):
        @pl.when(d < ndrain)
        def _drain():
            pltpu.make_async_copy(dummy_row, tmpc.at[pl.ds(0, EMBED_DIM)],
                                  sem_w).wait()
        return acc

    lax.fori_loop(0, _WRING, drain_body, jnp.int32(0), unroll=False)

    plsc.subcore_barrier()
    cbase = sid * _CPW

    def sub_body(sb, acc):
        coff = pl.multiple_of(cbase + sb * _TSUB, 128)
        pltpu.sync_copy(
            exch.at[pl.ds(obase + coff * EMBED_DIM, _TSUB * EMBED_DIM)],
            tbuf)

        def tr_d(d, acc):
            dsp = jnp.full((LANES,), d, jnp.int32)

            def tr_b(b, acc):
                cv = iota + b * LANES
                v = plsc.load_gather(tbuf, [cv * EMBED_DIM + d])
                plsc.store_scatter(colsT, [dsp, cv], v)
                return acc

            return lax.fori_loop(0, _TSUB // LANES, tr_b, acc,
                                 unroll=False)

        lax.fori_loop(0, EMBED_DIM, tr_d, jnp.int32(0), unroll=False)
        csl = pl.ds(coff, _TSUB)

        @pl.when(core == 0)
        def _wu():
            pltpu.sync_copy(colsT, uoutT.at[:, csl])

        @pl.when(core == 1)
        def _wi():
            pltpu.sync_copy(colsT, ioutT.at[:, csl])
        return acc

    lax.fori_loop(0, _CPW // _TSUB, sub_body, jnp.int32(0), unroll=False)


def kernel(user_ids, item_ids, user_table, item_table):
    ids_cat = jnp.concatenate([user_ids, item_ids])
    uT, iT, _ = _gather_kernel(ids_cat, user_table.T, item_table.T)
    return (uT.T, iT.T)
